# baseline (device time: 77558 ns/iter reference)
import jax
import jax.numpy as jnp
from jax import lax
from jax.experimental import pallas as pl
from jax.experimental.pallas import tpu as pltpu

N_CHUNKS = 16
N_LBLOCKS = 4


def kernel(x, dy):
    k, m = x.shape
    _, f = dy.shape
    half_m = m // 2
    f2 = f // 2
    cw = f2 // N_CHUNKS
    lb = f // N_LBLOCKS

    def body(
        x_ref,
        dy_ref,
        out_ref,
        s_buf,
        yrecv,
        xrecv,
        ysend_sems,
        yrecv_sems,
        fsend_sems,
        xrecv_sems,
    ):
        my_x = lax.axis_index("x")
        my_y = lax.axis_index("y")
        y_nbr = (my_x, 1 - my_y)
        x_nbr = (1 - my_x, my_y)

        barrier_sem = pltpu.get_barrier_semaphore()
        for nbr in (y_nbr, x_nbr):
            pl.semaphore_signal(
                barrier_sem,
                inc=1,
                device_id=nbr,
                device_id_type=pl.DeviceIdType.MESH,
            )
        pl.semaphore_wait(barrier_sem, 2)

        is_lo_y = my_y == 0
        is_x0 = my_x == 0
        x_mine = jnp.where(is_lo_y, x_ref[:, :half_m], x_ref[:, half_m:])
        x_other = jnp.where(is_lo_y, x_ref[:, half_m:], x_ref[:, :half_m])

        def dot(a, b):
            return lax.dot_general(
                a,
                b,
                dimension_numbers=(((0,), (0,)), ((), ())),
                preferred_element_type=jnp.float32,
            )

        y_rdmas = []
        for c in range(N_CHUNKS):
            lo, hi = c * cw, (c + 1) * cw

            @pl.when(is_x0)
            def _():
                s_buf[c] = dot(x_other, dy_ref[:, lo:hi])

            @pl.when(jnp.logical_not(is_x0))
            def _():
                s_buf[c] = dot(x_other, dy_ref[:, f2 + lo : f2 + hi])

            rdma = pltpu.make_async_remote_copy(
                src_ref=s_buf.at[c],
                dst_ref=yrecv.at[c],
                send_sem=ysend_sems.at[c],
                recv_sem=yrecv_sems.at[c],
                device_id=y_nbr,
                device_id_type=pl.DeviceIdType.MESH,
            )
            rdma.start()
            y_rdmas.append(rdma)

        f_rdmas = []
        per_block = N_CHUNKS // N_LBLOCKS
        for j in range(N_CHUNKS):
            y_rdmas[j].wait_recv()
            fwd = pltpu.make_async_remote_copy(
                src_ref=yrecv.at[j],
                dst_ref=xrecv.at[j],
                send_sem=fsend_sems.at[j],
                recv_sem=xrecv_sems.at[j],
                device_id=x_nbr,
                device_id_type=pl.DeviceIdType.MESH,
            )
            fwd.start()
            f_rdmas.append(fwd)
            if j % per_block == per_block - 1:
                blk = j // per_block
                blo, bhi = blk * lb, (blk + 1) * lb
                out_ref[:, blo:bhi] = dot(x_mine, dy_ref[:, blo:bhi])

        for j in range(N_CHUNKS):
            lo, hi = j * cw, (j + 1) * cw

            @pl.when(is_x0)
            def _():
                out_ref[:, lo:hi] = out_ref[:, lo:hi] + yrecv[j]

            @pl.when(jnp.logical_not(is_x0))
            def _():
                out_ref[:, f2 + lo : f2 + hi] = (
                    out_ref[:, f2 + lo : f2 + hi] + yrecv[j]
                )

        for j in range(N_CHUNKS):
            lo, hi = j * cw, (j + 1) * cw
            f_rdmas[j].wait_recv()

            @pl.when(is_x0)
            def _():
                out_ref[:, f2 + lo : f2 + hi] = (
                    out_ref[:, f2 + lo : f2 + hi] + xrecv[j]
                )

            @pl.when(jnp.logical_not(is_x0))
            def _():
                out_ref[:, lo:hi] = out_ref[:, lo:hi] + xrecv[j]

        for c in range(N_CHUNKS):
            y_rdmas[c].wait_send()
            f_rdmas[c].wait_send()

    return pl.pallas_call(
        body,
        out_shape=jax.ShapeDtypeStruct((half_m, f), jnp.float32),
        in_specs=[
            pl.BlockSpec(memory_space=pltpu.VMEM),
            pl.BlockSpec(memory_space=pltpu.VMEM),
        ],
        out_specs=pl.BlockSpec(memory_space=pltpu.VMEM),
        scratch_shapes=[
            pltpu.VMEM((N_CHUNKS, half_m, cw), jnp.float32),
            pltpu.VMEM((N_CHUNKS, half_m, cw), jnp.float32),
            pltpu.VMEM((N_CHUNKS, half_m, cw), jnp.float32),
            pltpu.SemaphoreType.DMA((N_CHUNKS,)),
            pltpu.SemaphoreType.DMA((N_CHUNKS,)),
            pltpu.SemaphoreType.DMA((N_CHUNKS,)),
            pltpu.SemaphoreType.DMA((N_CHUNKS,)),
        ],
        compiler_params=pltpu.CompilerParams(
            collective_id=0,
            vmem_limit_bytes=100 * 1024 * 1024,
        ),
    )(x, dy)


# device time: 50112 ns/iter; 1.5477x vs baseline; 1.5477x over previous
import jax
import jax.numpy as jnp
from jax import lax
from jax.experimental import pallas as pl
from jax.experimental.pallas import tpu as pltpu

N_CHUNKS = 16
LAG = 2
WIRE_DTYPE = jnp.bfloat16


def kernel(x, dy):
    k, m = x.shape
    _, f = dy.shape
    half_m = m // 2
    f2 = f // 2
    cw = f2 // N_CHUNKS

    def body(
        x_ref,
        dy_ref,
        out_ref,
        s_buf,
        yrecv,
        xrecv,
        ysend_sems,
        yrecv_sems,
        fsend_sems,
        xrecv_sems,
    ):
        my_x = lax.axis_index("x")
        my_y = lax.axis_index("y")
        y_nbr = (my_x, 1 - my_y)
        x_nbr = (1 - my_x, my_y)

        barrier_sem = pltpu.get_barrier_semaphore()
        for nbr in (y_nbr, x_nbr):
            pl.semaphore_signal(
                barrier_sem,
                inc=1,
                device_id=nbr,
                device_id_type=pl.DeviceIdType.MESH,
            )
        pl.semaphore_wait(barrier_sem, 2)

        is_lo_y = my_y == 0
        is_x0 = my_x == 0
        x_mine = jnp.where(is_lo_y, x_ref[:, :half_m], x_ref[:, half_m:])
        x_other = jnp.where(is_lo_y, x_ref[:, half_m:], x_ref[:, :half_m])

        def dot(a, b):
            return lax.dot_general(
                a,
                b,
                dimension_numbers=(((0,), (0,)), ((), ())),
                preferred_element_type=jnp.float32,
            )

        y_rdmas = []
        f_rdmas = []

        def consume_ychunk(j):
            lo, hi = j * cw, (j + 1) * cw
            y_rdmas[j].wait_recv()
            fwd = pltpu.make_async_remote_copy(
                src_ref=yrecv.at[j],
                dst_ref=xrecv.at[j],
                send_sem=fsend_sems.at[j],
                recv_sem=xrecv_sems.at[j],
                device_id=x_nbr,
                device_id_type=pl.DeviceIdType.MESH,
            )
            fwd.start()
            f_rdmas.append(fwd)

            @pl.when(is_x0)
            def _():
                out_ref[:, lo:hi] = out_ref[:, lo:hi] + yrecv[j].astype(
                    jnp.float32
                )

            @pl.when(jnp.logical_not(is_x0))
            def _():
                out_ref[:, f2 + lo : f2 + hi] = out_ref[
                    :, f2 + lo : f2 + hi
                ] + yrecv[j].astype(jnp.float32)

        for c in range(N_CHUNKS):
            lo, hi = c * cw, (c + 1) * cw

            @pl.when(is_x0)
            def _():
                s_buf[c] = dot(x_other, dy_ref[:, lo:hi]).astype(WIRE_DTYPE)

            @pl.when(jnp.logical_not(is_x0))
            def _():
                s_buf[c] = dot(
                    x_other, dy_ref[:, f2 + lo : f2 + hi]
                ).astype(WIRE_DTYPE)

            rdma = pltpu.make_async_remote_copy(
                src_ref=s_buf.at[c],
                dst_ref=yrecv.at[c],
                send_sem=ysend_sems.at[c],
                recv_sem=yrecv_sems.at[c],
                device_id=y_nbr,
                device_id_type=pl.DeviceIdType.MESH,
            )
            rdma.start()
            y_rdmas.append(rdma)

            out_ref[:, lo:hi] = dot(x_mine, dy_ref[:, lo:hi])
            out_ref[:, f2 + lo : f2 + hi] = dot(
                x_mine, dy_ref[:, f2 + lo : f2 + hi]
            )

            if c >= LAG:
                consume_ychunk(c - LAG)

        for j in range(N_CHUNKS - LAG, N_CHUNKS):
            consume_ychunk(j)

        for j in range(N_CHUNKS):
            lo, hi = j * cw, (j + 1) * cw
            f_rdmas[j].wait_recv()

            @pl.when(is_x0)
            def _():
                out_ref[:, f2 + lo : f2 + hi] = out_ref[
                    :, f2 + lo : f2 + hi
                ] + xrecv[j].astype(jnp.float32)

            @pl.when(jnp.logical_not(is_x0))
            def _():
                out_ref[:, lo:hi] = out_ref[:, lo:hi] + xrecv[j].astype(
                    jnp.float32
                )

        for c in range(N_CHUNKS):
            y_rdmas[c].wait_send()
            f_rdmas[c].wait_send()

    return pl.pallas_call(
        body,
        out_shape=jax.ShapeDtypeStruct((half_m, f), jnp.float32),
        in_specs=[
            pl.BlockSpec(memory_space=pltpu.VMEM),
            pl.BlockSpec(memory_space=pltpu.VMEM),
        ],
        out_specs=pl.BlockSpec(memory_space=pltpu.VMEM),
        scratch_shapes=[
            pltpu.VMEM((N_CHUNKS, half_m, cw), WIRE_DTYPE),
            pltpu.VMEM((N_CHUNKS, half_m, cw), WIRE_DTYPE),
            pltpu.VMEM((N_CHUNKS, half_m, cw), WIRE_DTYPE),
            pltpu.SemaphoreType.DMA((N_CHUNKS,)),
            pltpu.SemaphoreType.DMA((N_CHUNKS,)),
            pltpu.SemaphoreType.DMA((N_CHUNKS,)),
            pltpu.SemaphoreType.DMA((N_CHUNKS,)),
        ],
        compiler_params=pltpu.CompilerParams(
            collective_id=0,
            vmem_limit_bytes=100 * 1024 * 1024,
        ),
    )(x, dy)


# device time: 49810 ns/iter; 1.5571x vs baseline; 1.0061x over previous
import jax
import jax.numpy as jnp
from jax import lax
from jax.experimental import pallas as pl
from jax.experimental.pallas import tpu as pltpu

N_CHUNKS = 16
LAG = 2
WIRE_DTYPE = jnp.bfloat16


def kernel(x, dy):
    k, m = x.shape
    _, f = dy.shape
    half_m = m // 2
    f2 = f // 2
    cw = f2 // N_CHUNKS

    def body(
        x_ref,
        dy_ref,
        out_ref,
        s_buf,
        yrecv,
        xrecv,
        ysend_sems,
        yrecv_sems,
        fsend_sems,
        xrecv_sems,
    ):
        my_x = lax.axis_index("x")
        my_y = lax.axis_index("y")
        y_nbr = (my_x, 1 - my_y)
        x_nbr = (1 - my_x, my_y)

        barrier_sem = pltpu.get_barrier_semaphore()
        for nbr in (y_nbr, x_nbr):
            pl.semaphore_signal(
                barrier_sem,
                inc=1,
                device_id=nbr,
                device_id_type=pl.DeviceIdType.MESH,
            )
        pl.semaphore_wait(barrier_sem, 2)

        is_lo_y = my_y == 0
        is_x0 = my_x == 0
        x_mine = jnp.where(is_lo_y, x_ref[:, :half_m], x_ref[:, half_m:])
        x_other = jnp.where(is_lo_y, x_ref[:, half_m:], x_ref[:, :half_m])

        def dot(a, b):
            return lax.dot_general(
                a,
                b,
                dimension_numbers=(((0,), (0,)), ((), ())),
                preferred_element_type=jnp.float32,
            )

        y_rdmas = []
        f_rdmas = []

        def consume_ychunk(j):
            lo, hi = j * cw, (j + 1) * cw
            y_rdmas[j].wait_recv()
            fwd = pltpu.make_async_remote_copy(
                src_ref=yrecv.at[j],
                dst_ref=xrecv.at[j],
                send_sem=fsend_sems.at[j],
                recv_sem=xrecv_sems.at[j],
                device_id=x_nbr,
                device_id_type=pl.DeviceIdType.MESH,
            )
            fwd.start()
            f_rdmas.append(fwd)

            @pl.when(is_x0)
            def _():
                out_ref[:, lo:hi] = out_ref[:, lo:hi] + yrecv[j].astype(
                    jnp.float32
                )

            @pl.when(jnp.logical_not(is_x0))
            def _():
                out_ref[:, f2 + lo : f2 + hi] = out_ref[
                    :, f2 + lo : f2 + hi
                ] + yrecv[j].astype(jnp.float32)

        groups = N_CHUNKS // 4
        gw = 4 * cw
        for q in range(groups):
            glo, ghi = q * gw, (q + 1) * gw

            @pl.when(is_x0)
            def _():
                blk = dot(x_other, dy_ref[:, glo:ghi]).astype(WIRE_DTYPE)
                for i in range(4):
                    s_buf[4 * q + i] = blk[:, i * cw : (i + 1) * cw]

            @pl.when(jnp.logical_not(is_x0))
            def _():
                blk = dot(
                    x_other, dy_ref[:, f2 + glo : f2 + ghi]
                ).astype(WIRE_DTYPE)
                for i in range(4):
                    s_buf[4 * q + i] = blk[:, i * cw : (i + 1) * cw]

            for i in range(4):
                c = 4 * q + i
                rdma = pltpu.make_async_remote_copy(
                    src_ref=s_buf.at[c],
                    dst_ref=yrecv.at[c],
                    send_sem=ysend_sems.at[c],
                    recv_sem=yrecv_sems.at[c],
                    device_id=y_nbr,
                    device_id_type=pl.DeviceIdType.MESH,
                )
                rdma.start()
                y_rdmas.append(rdma)

        for q in range(groups):
            glo, ghi = q * gw, (q + 1) * gw
            out_ref[:, glo:ghi] = dot(x_mine, dy_ref[:, glo:ghi])
            out_ref[:, f2 + glo : f2 + ghi] = dot(
                x_mine, dy_ref[:, f2 + glo : f2 + ghi]
            )
            for j in range(4 * q, 4 * q + 4):
                consume_ychunk(j)

        for j in range(N_CHUNKS):
            lo, hi = j * cw, (j + 1) * cw
            f_rdmas[j].wait_recv()

            @pl.when(is_x0)
            def _():
                out_ref[:, f2 + lo : f2 + hi] = out_ref[
                    :, f2 + lo : f2 + hi
                ] + xrecv[j].astype(jnp.float32)

            @pl.when(jnp.logical_not(is_x0))
            def _():
                out_ref[:, lo:hi] = out_ref[:, lo:hi] + xrecv[j].astype(
                    jnp.float32
                )

        for c in range(N_CHUNKS):
            y_rdmas[c].wait_send()
            f_rdmas[c].wait_send()

    return pl.pallas_call(
        body,
        out_shape=jax.ShapeDtypeStruct((half_m, f), jnp.float32),
        in_specs=[
            pl.BlockSpec(memory_space=pltpu.VMEM),
            pl.BlockSpec(memory_space=pltpu.VMEM),
        ],
        out_specs=pl.BlockSpec(memory_space=pltpu.VMEM),
        scratch_shapes=[
            pltpu.VMEM((N_CHUNKS, half_m, cw), WIRE_DTYPE),
            pltpu.VMEM((N_CHUNKS, half_m, cw), WIRE_DTYPE),
            pltpu.VMEM((N_CHUNKS, half_m, cw), WIRE_DTYPE),
            pltpu.SemaphoreType.DMA((N_CHUNKS,)),
            pltpu.SemaphoreType.DMA((N_CHUNKS,)),
            pltpu.SemaphoreType.DMA((N_CHUNKS,)),
            pltpu.SemaphoreType.DMA((N_CHUNKS,)),
        ],
        compiler_params=pltpu.CompilerParams(
            collective_id=0,
            vmem_limit_bytes=100 * 1024 * 1024,
        ),
    )(x, dy)


# device time: 46859 ns/iter; 1.6551x vs baseline; 1.0630x over previous
import jax
import jax.numpy as jnp
from jax import lax
from jax.experimental import pallas as pl
from jax.experimental.pallas import tpu as pltpu

N_CHUNKS = 16
WIRE_DTYPE = jnp.bfloat16


def kernel(x, dy):
    k, m = x.shape
    _, f = dy.shape
    half_m = m // 2
    f2 = f // 2
    cw = f2 // N_CHUNKS
    groups = N_CHUNKS // 4
    gw = 4 * cw

    def body(
        x_ref,
        dy_hbm,
        out_ref,
        dy_vmem,
        s_buf,
        yrecv,
        xrecv,
        copy_sems,
        ysend_sems,
        yrecv_sems,
        fsend_sems,
        xrecv_sems,
    ):
        my_x = lax.axis_index("x")
        my_y = lax.axis_index("y")
        y_nbr = (my_x, 1 - my_y)
        x_nbr = (1 - my_x, my_y)

        dy_copies = [None] * (2 * groups)
        for q in range(groups):
            for blk in (q, groups + q):
                blo, bhi = blk * gw, (blk + 1) * gw
                cp = pltpu.make_async_copy(
                    dy_hbm.at[:, blo:bhi],
                    dy_vmem.at[:, blo:bhi],
                    copy_sems.at[blk],
                )
                cp.start()
                dy_copies[blk] = cp

        barrier_sem = pltpu.get_barrier_semaphore()
        for nbr in (y_nbr, x_nbr):
            pl.semaphore_signal(
                barrier_sem,
                inc=1,
                device_id=nbr,
                device_id_type=pl.DeviceIdType.MESH,
            )
        pl.semaphore_wait(barrier_sem, 2)

        is_lo_y = my_y == 0
        is_x0 = my_x == 0
        x_mine = jnp.where(is_lo_y, x_ref[:, :half_m], x_ref[:, half_m:])
        x_other = jnp.where(is_lo_y, x_ref[:, half_m:], x_ref[:, :half_m])

        def dot(a, b):
            return lax.dot_general(
                a,
                b,
                dimension_numbers=(((0,), (0,)), ((), ())),
                preferred_element_type=jnp.float32,
            )

        y_rdmas = []
        f_rdmas = []

        def consume_ychunk(j):
            lo, hi = j * cw, (j + 1) * cw
            y_rdmas[j].wait_recv()
            fwd = pltpu.make_async_remote_copy(
                src_ref=yrecv.at[j],
                dst_ref=xrecv.at[j],
                send_sem=fsend_sems.at[j],
                recv_sem=xrecv_sems.at[j],
                device_id=x_nbr,
                device_id_type=pl.DeviceIdType.MESH,
            )
            fwd.start()
            f_rdmas.append(fwd)

            @pl.when(is_x0)
            def _():
                out_ref[:, lo:hi] = out_ref[:, lo:hi] + yrecv[j].astype(
                    jnp.float32
                )

            @pl.when(jnp.logical_not(is_x0))
            def _():
                out_ref[:, f2 + lo : f2 + hi] = out_ref[
                    :, f2 + lo : f2 + hi
                ] + yrecv[j].astype(jnp.float32)

        for q in range(groups):
            glo, ghi = q * gw, (q + 1) * gw

            @pl.when(is_x0)
            def _():
                dy_copies[q].wait()
                blk = dot(x_other, dy_vmem[:, glo:ghi]).astype(WIRE_DTYPE)
                for i in range(4):
                    s_buf[4 * q + i] = blk[:, i * cw : (i + 1) * cw]

            @pl.when(jnp.logical_not(is_x0))
            def _():
                dy_copies[groups + q].wait()
                blk = dot(
                    x_other, dy_vmem[:, f2 + glo : f2 + ghi]
                ).astype(WIRE_DTYPE)
                for i in range(4):
                    s_buf[4 * q + i] = blk[:, i * cw : (i + 1) * cw]

            for i in range(4):
                c = 4 * q + i
                rdma = pltpu.make_async_remote_copy(
                    src_ref=s_buf.at[c],
                    dst_ref=yrecv.at[c],
                    send_sem=ysend_sems.at[c],
                    recv_sem=yrecv_sems.at[c],
                    device_id=y_nbr,
                    device_id_type=pl.DeviceIdType.MESH,
                )
                rdma.start()
                y_rdmas.append(rdma)

        for q in range(groups):
            glo, ghi = q * gw, (q + 1) * gw

            @pl.when(is_x0)
            def _():
                dy_copies[groups + q].wait()

            @pl.when(jnp.logical_not(is_x0))
            def _():
                dy_copies[q].wait()

            out_ref[:, glo:ghi] = dot(x_mine, dy_vmem[:, glo:ghi])
            out_ref[:, f2 + glo : f2 + ghi] = dot(
                x_mine, dy_vmem[:, f2 + glo : f2 + ghi]
            )
            for j in range(4 * q, 4 * q + 4):
                consume_ychunk(j)

        for j in range(N_CHUNKS):
            lo, hi = j * cw, (j + 1) * cw
            f_rdmas[j].wait_recv()

            @pl.when(is_x0)
            def _():
                out_ref[:, f2 + lo : f2 + hi] = out_ref[
                    :, f2 + lo : f2 + hi
                ] + xrecv[j].astype(jnp.float32)

            @pl.when(jnp.logical_not(is_x0))
            def _():
                out_ref[:, lo:hi] = out_ref[:, lo:hi] + xrecv[j].astype(
                    jnp.float32
                )

        for c in range(N_CHUNKS):
            y_rdmas[c].wait_send()
            f_rdmas[c].wait_send()

    return pl.pallas_call(
        body,
        out_shape=jax.ShapeDtypeStruct((half_m, f), jnp.float32),
        in_specs=[
            pl.BlockSpec(memory_space=pltpu.VMEM),
            pl.BlockSpec(memory_space=pl.ANY),
        ],
        out_specs=pl.BlockSpec(memory_space=pltpu.VMEM),
        scratch_shapes=[
            pltpu.VMEM((k, f), jnp.float32),
            pltpu.VMEM((N_CHUNKS, half_m, cw), WIRE_DTYPE),
            pltpu.VMEM((N_CHUNKS, half_m, cw), WIRE_DTYPE),
            pltpu.VMEM((N_CHUNKS, half_m, cw), WIRE_DTYPE),
            pltpu.SemaphoreType.DMA((2 * (N_CHUNKS // 4),)),
            pltpu.SemaphoreType.DMA((N_CHUNKS,)),
            pltpu.SemaphoreType.DMA((N_CHUNKS,)),
            pltpu.SemaphoreType.DMA((N_CHUNKS,)),
            pltpu.SemaphoreType.DMA((N_CHUNKS,)),
        ],
        compiler_params=pltpu.CompilerParams(
            collective_id=0,
            vmem_limit_bytes=100 * 1024 * 1024,
        ),
    )(x, dy)


# device time: 44868 ns/iter; 1.7286x vs baseline; 1.0444x over previous
import jax
import jax.numpy as jnp
from jax import lax
from jax.experimental import pallas as pl
from jax.experimental.pallas import tpu as pltpu

N_CHUNKS = 16
WIRE_DTYPE = jnp.bfloat16


def kernel(x, dy):
    k, m = x.shape
    _, f = dy.shape
    half_m = m // 2
    f2 = f // 2
    cw = f2 // N_CHUNKS
    groups = N_CHUNKS // 4
    gw = 4 * cw

    def body(
        x_ref,
        dy_hbm,
        out_hbm,
        dy_vmem,
        out_ref,
        s_buf,
        yrecv,
        xrecv,
        copy_sems,
        out_sems,
        ysend_sems,
        yrecv_sems,
        fsend_sems,
        xrecv_sems,
    ):
        my_x = lax.axis_index("x")
        my_y = lax.axis_index("y")
        y_nbr = (my_x, 1 - my_y)
        x_nbr = (1 - my_x, my_y)

        dy_copies = [None] * (2 * groups)
        for q in range(groups):
            for blk in (q, groups + q):
                blo, bhi = blk * gw, (blk + 1) * gw
                cp = pltpu.make_async_copy(
                    dy_hbm.at[:, blo:bhi],
                    dy_vmem.at[:, blo:bhi],
                    copy_sems.at[blk],
                )
                cp.start()
                dy_copies[blk] = cp

        barrier_sem = pltpu.get_barrier_semaphore()
        for nbr in (y_nbr, x_nbr):
            pl.semaphore_signal(
                barrier_sem,
                inc=1,
                device_id=nbr,
                device_id_type=pl.DeviceIdType.MESH,
            )
        pl.semaphore_wait(barrier_sem, 2)

        is_lo_y = my_y == 0
        is_x0 = my_x == 0
        x_mine = jnp.where(is_lo_y, x_ref[:, :half_m], x_ref[:, half_m:])
        x_other = jnp.where(is_lo_y, x_ref[:, half_m:], x_ref[:, :half_m])

        def dot(a, b):
            return lax.dot_general(
                a,
                b,
                dimension_numbers=(((0,), (0,)), ((), ())),
                preferred_element_type=jnp.float32,
            )

        y_rdmas = []
        f_rdmas = []

        def consume_ychunk(j):
            lo, hi = j * cw, (j + 1) * cw
            y_rdmas[j].wait_recv()
            fwd = pltpu.make_async_remote_copy(
                src_ref=yrecv.at[j],
                dst_ref=xrecv.at[j],
                send_sem=fsend_sems.at[j],
                recv_sem=xrecv_sems.at[j],
                device_id=x_nbr,
                device_id_type=pl.DeviceIdType.MESH,
            )
            fwd.start()
            f_rdmas.append(fwd)

            @pl.when(is_x0)
            def _():
                out_ref[:, lo:hi] = out_ref[:, lo:hi] + yrecv[j].astype(
                    jnp.float32
                )

            @pl.when(jnp.logical_not(is_x0))
            def _():
                out_ref[:, f2 + lo : f2 + hi] = out_ref[
                    :, f2 + lo : f2 + hi
                ] + yrecv[j].astype(jnp.float32)

        for q in range(groups):
            glo, ghi = q * gw, (q + 1) * gw

            @pl.when(is_x0)
            def _():
                dy_copies[q].wait()
                blk = dot(x_other, dy_vmem[:, glo:ghi]).astype(WIRE_DTYPE)
                for i in range(4):
                    s_buf[4 * q + i] = blk[:, i * cw : (i + 1) * cw]

            @pl.when(jnp.logical_not(is_x0))
            def _():
                dy_copies[groups + q].wait()
                blk = dot(
                    x_other, dy_vmem[:, f2 + glo : f2 + ghi]
                ).astype(WIRE_DTYPE)
                for i in range(4):
                    s_buf[4 * q + i] = blk[:, i * cw : (i + 1) * cw]

            for i in range(4):
                c = 4 * q + i
                rdma = pltpu.make_async_remote_copy(
                    src_ref=s_buf.at[c],
                    dst_ref=yrecv.at[c],
                    send_sem=ysend_sems.at[c],
                    recv_sem=yrecv_sems.at[c],
                    device_id=y_nbr,
                    device_id_type=pl.DeviceIdType.MESH,
                )
                rdma.start()
                y_rdmas.append(rdma)

        for q in range(groups):
            glo, ghi = q * gw, (q + 1) * gw

            @pl.when(is_x0)
            def _():
                dy_copies[groups + q].wait()

            @pl.when(jnp.logical_not(is_x0))
            def _():
                dy_copies[q].wait()

            out_ref[:, glo:ghi] = dot(x_mine, dy_vmem[:, glo:ghi])
            out_ref[:, f2 + glo : f2 + ghi] = dot(
                x_mine, dy_vmem[:, f2 + glo : f2 + ghi]
            )
            for j in range(4 * q, 4 * q + 4):
                consume_ychunk(j)

            @pl.when(is_x0)
            def _():
                pltpu.make_async_copy(
                    out_ref.at[:, glo:ghi],
                    out_hbm.at[:, glo:ghi],
                    out_sems.at[q],
                ).start()

            @pl.when(jnp.logical_not(is_x0))
            def _():
                pltpu.make_async_copy(
                    out_ref.at[:, f2 + glo : f2 + ghi],
                    out_hbm.at[:, f2 + glo : f2 + ghi],
                    out_sems.at[q],
                ).start()

        for q in range(groups):
            glo, ghi = q * gw, (q + 1) * gw
            for j in range(4 * q, 4 * q + 4):
                lo, hi = j * cw, (j + 1) * cw
                f_rdmas[j].wait_recv()

                @pl.when(is_x0)
                def _():
                    out_ref[:, f2 + lo : f2 + hi] = out_ref[
                        :, f2 + lo : f2 + hi
                    ] + xrecv[j].astype(jnp.float32)

                @pl.when(jnp.logical_not(is_x0))
                def _():
                    out_ref[:, lo:hi] = out_ref[:, lo:hi] + xrecv[
                        j
                    ].astype(jnp.float32)

            @pl.when(is_x0)
            def _():
                pltpu.make_async_copy(
                    out_ref.at[:, f2 + glo : f2 + ghi],
                    out_hbm.at[:, f2 + glo : f2 + ghi],
                    out_sems.at[groups + q],
                ).start()

            @pl.when(jnp.logical_not(is_x0))
            def _():
                pltpu.make_async_copy(
                    out_ref.at[:, glo:ghi],
                    out_hbm.at[:, glo:ghi],
                    out_sems.at[groups + q],
                ).start()

        for q in range(2 * groups):
            blo, bhi = (q % groups) * gw, (q % groups + 1) * gw
            pltpu.make_async_copy(
                out_ref.at[:, blo:bhi],
                out_hbm.at[:, blo:bhi],
                out_sems.at[q],
            ).wait()

        for c in range(N_CHUNKS):
            y_rdmas[c].wait_send()
            f_rdmas[c].wait_send()

    return pl.pallas_call(
        body,
        out_shape=jax.ShapeDtypeStruct((half_m, f), jnp.float32),
        in_specs=[
            pl.BlockSpec(memory_space=pltpu.VMEM),
            pl.BlockSpec(memory_space=pl.ANY),
        ],
        out_specs=pl.BlockSpec(memory_space=pl.ANY),
        scratch_shapes=[
            pltpu.VMEM((k, f), jnp.float32),
            pltpu.VMEM((m // 2, f), jnp.float32),
            pltpu.VMEM((N_CHUNKS, half_m, cw), WIRE_DTYPE),
            pltpu.VMEM((N_CHUNKS, half_m, cw), WIRE_DTYPE),
            pltpu.VMEM((N_CHUNKS, half_m, cw), WIRE_DTYPE),
            pltpu.SemaphoreType.DMA((2 * (N_CHUNKS // 4),)),
            pltpu.SemaphoreType.DMA((2 * (N_CHUNKS // 4),)),
            pltpu.SemaphoreType.DMA((N_CHUNKS,)),
            pltpu.SemaphoreType.DMA((N_CHUNKS,)),
            pltpu.SemaphoreType.DMA((N_CHUNKS,)),
            pltpu.SemaphoreType.DMA((N_CHUNKS,)),
        ],
        compiler_params=pltpu.CompilerParams(
            collective_id=0,
            vmem_limit_bytes=100 * 1024 * 1024,
        ),
    )(x, dy)
